# baseline (speedup 1.0000x reference)
import jax
import jax.numpy as jnp
from jax.experimental import pallas as pl
from jax.experimental.pallas import tpu as pltpu

_LANE = 128
_TBP = 256


def _round_up(x, m):
    return ((x + m - 1) // m) * m


def _mlp_packed_kernel(xp_ref, wp_ref, b1p_ref, w2p_ref, b2p_ref, out_ref):
    xp = xp_ref[...]
    hp = jnp.dot(xp, wp_ref[...], preferred_element_type=jnp.float32)
    hp = jnp.maximum(hp + b1p_ref[...], 0.0)
    op = jnp.dot(hp, w2p_ref[...], preferred_element_type=jnp.float32)
    out_ref[...] = op + b2p_ref[...]


def kernel(x, w1, b1, w2_p, b2_p):
    batch, obs = x.shape
    hidden = w1.shape[1]
    n_actions = 2
    pack = _LANE // obs

    rows = _round_up(batch, pack * _TBP) // pack
    tbp = min(_TBP, rows)
    b_pad = rows * pack
    x_p = x if b_pad == batch else jnp.pad(x, ((0, b_pad - batch), (0, 0)))
    xp = x_p.reshape(rows, _LANE)

    eye = jnp.eye(pack, dtype=x.dtype)
    wp = (eye[:, None, :, None] * w1[None, :, None, :]).reshape(
        pack * obs, pack * hidden)
    b1p = jnp.tile(b1, (1, pack))
    w2 = w2_p[:, :n_actions]
    w2p = (eye[:, None, :, None] * w2[None, :, None, :]).reshape(
        pack * hidden, pack * n_actions)
    b2p = jnp.tile(b2_p[:, :n_actions], (1, pack))

    grid = (rows // tbp,)
    out = pl.pallas_call(
        _mlp_packed_kernel,
        out_shape=jax.ShapeDtypeStruct((rows, pack * n_actions), jnp.float32),
        grid=grid,
        in_specs=[
            pl.BlockSpec((tbp, _LANE), lambda i: (i, 0)),
            pl.BlockSpec((pack * obs, pack * hidden), lambda i: (0, 0)),
            pl.BlockSpec((1, pack * hidden), lambda i: (0, 0)),
            pl.BlockSpec((pack * hidden, pack * n_actions), lambda i: (0, 0)),
            pl.BlockSpec((1, pack * n_actions), lambda i: (0, 0)),
        ],
        out_specs=pl.BlockSpec((tbp, pack * n_actions), lambda i: (i, 0)),
        compiler_params=pltpu.CompilerParams(
            dimension_semantics=("parallel",)),
    )(xp, wp, b1p, w2p, b2p)

    return out.reshape(b_pad, n_actions)[:batch]


# speedup vs baseline: 2.1246x; 2.1246x over previous
import jax
import jax.numpy as jnp
from jax.experimental import pallas as pl
from jax.experimental.pallas import tpu as pltpu

_TB = 2048


def _round_up(x, m):
    return ((x + m - 1) // m) * m


def _mlp_kernel(x_ref, w1_ref, b1_ref, w2_ref, b2_ref, out_ref):
    x = x_ref[...]
    h = jnp.dot(x, w1_ref[...], preferred_element_type=jnp.float32)
    h = jnp.maximum(h + b1_ref[...], 0.0)
    o = jnp.dot(h, w2_ref[...], preferred_element_type=jnp.float32)
    out_ref[...] = o[:, :out_ref.shape[1]] + b2_ref[...]


def kernel(x, w1, b1, w2_p, b2_p):
    batch, obs = x.shape
    hidden = w1.shape[1]
    n_actions = 2

    b_pad = _round_up(batch, _TB)
    tb = min(_TB, b_pad)
    x_p = x if b_pad == batch else jnp.pad(x, ((0, b_pad - batch), (0, 0)))

    w2 = w2_p[:, :n_actions]
    b2 = b2_p[:, :n_actions]

    grid = (b_pad // tb,)
    out = pl.pallas_call(
        _mlp_kernel,
        out_shape=jax.ShapeDtypeStruct((b_pad, n_actions), jnp.float32),
        grid=grid,
        in_specs=[
            pl.BlockSpec((tb, obs), lambda i: (i, 0)),
            pl.BlockSpec((obs, hidden), lambda i: (0, 0)),
            pl.BlockSpec((1, hidden), lambda i: (0, 0)),
            pl.BlockSpec((hidden, n_actions), lambda i: (0, 0)),
            pl.BlockSpec((1, n_actions), lambda i: (0, 0)),
        ],
        out_specs=pl.BlockSpec((tb, n_actions), lambda i: (i, 0)),
        compiler_params=pltpu.CompilerParams(
            dimension_semantics=("parallel",)),
    )(x_p, w1, b1, w2, b2)

    return out[:batch]


# speedup vs baseline: 17.3016x; 8.1433x over previous
import jax
import jax.numpy as jnp
from jax.experimental import pallas as pl
from jax.experimental.pallas import tpu as pltpu

_NT = 8192


def _round_up(x, m):
    return ((x + m - 1) // m) * m


def _mlp_t_kernel(xt_ref, w1_ref, b1t_ref, w2_ref, b2t_ref, out_ref):
    xt = xt_ref[...]
    ht = jax.lax.dot_general(
        w1_ref[...], xt, (((0,), (0,)), ((), ())),
        preferred_element_type=jnp.float32)
    ht = jnp.maximum(ht + b1t_ref[...], 0.0)
    ot = jax.lax.dot_general(
        w2_ref[...], ht, (((0,), (0,)), ((), ())),
        preferred_element_type=jnp.float32)
    out_ref[...] = ot + b2t_ref[...]


def kernel(x, w1, b1, w2_p, b2_p):
    batch, obs = x.shape
    hidden = w1.shape[1]
    n_actions = 2

    b_pad = _round_up(batch, _NT)
    nt = min(_NT, b_pad)
    x_p = x if b_pad == batch else jnp.pad(x, ((0, b_pad - batch), (0, 0)))

    xt = x_p.T
    b1t = b1.T
    w2 = w2_p[:, :n_actions]
    b2t = b2_p[:, :n_actions].T

    grid = (b_pad // nt,)
    out_t = pl.pallas_call(
        _mlp_t_kernel,
        out_shape=jax.ShapeDtypeStruct((n_actions, b_pad), jnp.float32),
        grid=grid,
        in_specs=[
            pl.BlockSpec((obs, nt), lambda i: (0, i)),
            pl.BlockSpec((obs, hidden), lambda i: (0, 0)),
            pl.BlockSpec((hidden, 1), lambda i: (0, 0)),
            pl.BlockSpec((hidden, n_actions), lambda i: (0, 0)),
            pl.BlockSpec((n_actions, 1), lambda i: (0, 0)),
        ],
        out_specs=pl.BlockSpec((n_actions, nt), lambda i: (0, i)),
        compiler_params=pltpu.CompilerParams(
            dimension_semantics=("parallel",)),
    )(xt, w1, b1t, w2, b2t)

    return out_t[:, :batch].T
